# baseline (speedup 1.0000x reference)
import functools

import jax
import jax.numpy as jnp
from jax import lax
from jax.experimental import pallas as pl
from jax.experimental.pallas import tpu as pltpu
from jax.experimental.pallas import tpu_sc as plsc

NUM_Q = 100000
EMB = 64
PADW = 128

NC = 2
NS = 16
L = 16
NW = NC * NS


def _make_kernel(T: int, NBATCH: int):
    LB = NBATCH // NW
    assert LB % L == 0 and EMB % 8 == 0 and T % 2 == 0
    mesh = plsc.VectorSubcoreMesh(core_axis_name="c", subcore_axis_name="s")

    @functools.partial(
        pl.kernel,
        mesh=mesh,
        compiler_params=pltpu.CompilerParams(
            use_tc_tiling_on_sc=True, needs_layout_passes=False),
        out_type=jax.ShapeDtypeStruct((T, EMB, NBATCH), jnp.float32),
        scratch_types=[
            pltpu.VMEM((T, LB), jnp.int32),
            pltpu.VMEM((T, LB), jnp.int32),
            pltpu.VMEM((LB, PADW), jnp.float32),
            pltpu.VMEM((LB, PADW), jnp.float32),
            pltpu.VMEM((EMB, LB), jnp.float32),
            pltpu.VMEM((EMB, LB), jnp.float32),
            pltpu.SemaphoreType.DMA,
            pltpu.SemaphoreType.DMA,
            pltpu.SemaphoreType.DMA,
            pltpu.SemaphoreType.DMA,
        ],
    )
    def gather_kernel(qT, rT, tab, outT,
                      qv, rv, row0, row1, tile0, tile1, g0, g1, o0, o1):
        wid = lax.axis_index("s") * NC + lax.axis_index("c")
        lane0 = wid * LB

        pltpu.sync_copy(qT.at[:, pl.ds(lane0, LB)], qv)
        pltpu.sync_copy(rT.at[:, pl.ds(lane0, LB)], rv)

        @pl.loop(0, T)
        def _idx(t):
            for s in range(LB // L):
                sl = pl.ds(s * L, L)
                qv[t, sl] = qv[t, sl] + NUM_Q * rv[t, sl]

        iot = lax.iota(jnp.int32, L)

        def fire_gather(t, row, g):
            pltpu.async_copy(tab.at[qv.at[t]], row, g)

        def drain_gather(row, g):
            pltpu.make_async_copy(tab.at[pl.ds(0, LB)], row, g).wait()

        def transpose(row, tile):
            @pl.loop(0, EMB, unroll=8)
            def _e(e):
                ce = jnp.full((L,), e, jnp.int32)
                for j in range(LB // L):
                    vals = plsc.load_gather(row, [j * L + iot, ce])
                    tile[e, pl.ds(j * L, L)] = vals

        def fire_store(t, tile, o):
            pltpu.async_copy(tile, outT.at[t, :, pl.ds(lane0, LB)], o)

        def drain_store(tile, o):
            pltpu.make_async_copy(tile, outT.at[0, :, pl.ds(0, LB)], o).wait()

        fire_gather(0, row0, g0)

        @pl.loop(0, T // 2)
        def _main(i):
            t0 = 2 * i
            fire_gather(t0 + 1, row1, g1)
            drain_gather(row0, g0)

            @pl.when(i > 0)
            def _():
                drain_store(tile0, o0)

            transpose(row0, tile0)

            @pl.when(i < T // 2 - 1)
            def _():
                fire_gather(t0 + 2, row0, g0)

            fire_store(t0, tile0, o0)
            drain_gather(row1, g1)

            @pl.when(i > 0)
            def _():
                drain_store(tile1, o1)

            transpose(row1, tile1)
            fire_store(t0 + 1, tile1, o1)

        drain_store(tile0, o0)
        drain_store(tile1, o1)

    return gather_kernel


def kernel(q, r, interaction_emb):
    nbatch, t = q.shape
    qT = q.T.astype(jnp.int32)
    rT = r.T.astype(jnp.int32)
    tab = jnp.pad(interaction_emb, ((0, 0), (0, PADW - EMB)))
    outT = _make_kernel(t, nbatch)(qT, rT, tab)
    return outT.transpose(2, 0, 1)


# speedup vs baseline: 1.2056x; 1.2056x over previous
import functools

import jax
import jax.numpy as jnp
from jax import lax
from jax.experimental import pallas as pl
from jax.experimental.pallas import tpu as pltpu
from jax.experimental.pallas import tpu_sc as plsc

NUM_Q = 100000
EMB = 64
PADW = 128

NC = 2
NS = 16
L = 16
NW = NC * NS


def _make_kernel(T: int, NBATCH: int):
    LB = NBATCH // NW
    assert LB % L == 0 and EMB % 8 == 0 and T % 2 == 0
    mesh = plsc.VectorSubcoreMesh(core_axis_name="c", subcore_axis_name="s")

    @functools.partial(
        pl.kernel,
        mesh=mesh,
        compiler_params=pltpu.CompilerParams(
            use_tc_tiling_on_sc=True, needs_layout_passes=False),
        out_type=jax.ShapeDtypeStruct((T, EMB, NBATCH), jnp.float32),
        scratch_types=[
            pltpu.VMEM((T, LB), jnp.int32),
            pltpu.VMEM((T, LB), jnp.int32),
            pltpu.VMEM((LB, PADW), jnp.float32),
            pltpu.VMEM((LB, PADW), jnp.float32),
            pltpu.VMEM((EMB, LB), jnp.float32),
            pltpu.VMEM((EMB, LB), jnp.float32),
            pltpu.SemaphoreType.DMA,
            pltpu.SemaphoreType.DMA,
            pltpu.SemaphoreType.DMA,
            pltpu.SemaphoreType.DMA,
        ],
    )
    def gather_kernel(qT, rT, tab, outT,
                      qv, rv, row0, row1, tile0, tile1, g0, g1, o0, o1):
        wid = lax.axis_index("s") * NC + lax.axis_index("c")
        lane0 = wid * LB

        pltpu.sync_copy(qT.at[:, pl.ds(lane0, LB)], qv)
        pltpu.sync_copy(rT.at[:, pl.ds(lane0, LB)], rv)

        @pl.loop(0, T)
        def _idx(t):
            for s in range(LB // L):
                sl = pl.ds(s * L, L)
                qv[t, sl] = qv[t, sl] + NUM_Q * rv[t, sl]

        iot = lax.iota(jnp.int32, L)

        def fire_gather(t, row, g):
            pltpu.async_copy(tab.at[qv.at[t]], row, g)

        def drain_gather(row, g):
            pltpu.make_async_copy(tab.at[pl.ds(0, LB)], row, g).wait()

        rjs = [j * L + iot for j in range(LB // L)]

        def transpose(row, tile):
            @pl.loop(0, EMB, unroll=2)
            def _e(e):
                ce = jnp.full((L,), e, jnp.int32)
                vals = [plsc.load_gather(row, [rj, ce]) for rj in rjs]
                for j, v in enumerate(vals):
                    tile[e, pl.ds(j * L, L)] = v

        def fire_store(t, tile, o):
            pltpu.async_copy(tile, outT.at[t, :, pl.ds(lane0, LB)], o)

        def drain_store(tile, o):
            pltpu.make_async_copy(tile, outT.at[0, :, pl.ds(0, LB)], o).wait()

        fire_gather(0, row0, g0)

        @pl.loop(0, T // 2)
        def _main(i):
            t0 = 2 * i
            fire_gather(t0 + 1, row1, g1)
            drain_gather(row0, g0)

            @pl.when(i > 0)
            def _():
                drain_store(tile0, o0)

            transpose(row0, tile0)

            @pl.when(i < T // 2 - 1)
            def _():
                fire_gather(t0 + 2, row0, g0)

            fire_store(t0, tile0, o0)
            drain_gather(row1, g1)

            @pl.when(i > 0)
            def _():
                drain_store(tile1, o1)

            transpose(row1, tile1)
            fire_store(t0 + 1, tile1, o1)

        drain_store(tile0, o0)
        drain_store(tile1, o1)

    return gather_kernel


def kernel(q, r, interaction_emb):
    nbatch, t = q.shape
    qT = q.T.astype(jnp.int32)
    rT = r.T.astype(jnp.int32)
    tab = jnp.pad(interaction_emb, ((0, 0), (0, PADW - EMB)))
    outT = _make_kernel(t, nbatch)(qT, rT, tab)
    return outT.transpose(2, 0, 1)
